# Initial kernel scaffold; baseline (speedup 1.0000x reference)
import jax
import jax.numpy as jnp
from jax.experimental import pallas as pl


def kernel(x, edge_index):
    raise NotImplementedError("write your pallas kernel here")


# speedup vs baseline: 16.7067x; 16.7067x over previous
import functools

import jax
import jax.numpy as jnp
from jax import lax
from jax.experimental import pallas as pl
from jax.experimental.pallas import tpu as pltpu
from jax.experimental.pallas import tpu_sc as plsc

_N = 1024
_E = 16384

_NC = 2
_NS = 16
_NW = _NC * _NS
_ROWS = _N // _NW
_L = 16

@functools.lru_cache(maxsize=None)
def _build_adj_kernel():
    mesh = plsc.VectorSubcoreMesh(core_axis_name="c", subcore_axis_name="s")

    @functools.partial(
        pl.kernel,
        mesh=mesh,
        out_type=jax.ShapeDtypeStruct((_N * _N,), jnp.float32),
        scratch_types=[
            pltpu.VMEM((_E,), jnp.int32),
            pltpu.VMEM((_E,), jnp.int32),
            pltpu.VMEM((_ROWS * _N,), jnp.float32),
        ],
        compiler_params=pltpu.CompilerParams(needs_layout_passes=False),
    )
    def _build_adj(edge_hbm, out_hbm, src_v, dst_v, buf_v):
        wid = lax.axis_index("s") * _NC + lax.axis_index("c")
        base = wid * _ROWS

        pltpu.sync_copy(edge_hbm.at[0], src_v)
        pltpu.sync_copy(edge_hbm.at[1], dst_v)

        zeros = jnp.zeros((_L,), jnp.float32)

        def _zero(i, carry):
            buf_v[pl.ds(i * _L, _L)] = zeros
            return carry

        lax.fori_loop(0, (_ROWS * _N) // _L, _zero, 0)

        ones = jnp.ones((_L,), jnp.float32)

        def _scatter(i, carry):
            s16 = src_v[pl.ds(i * _L, _L)]
            d16 = dst_v[pl.ds(i * _L, _L)]
            rel = d16 - base
            m = (rel >= 0) & (rel < _ROWS)
            lin = jnp.where(m, rel * _N + s16, 0)
            plsc.store_scatter(buf_v, [lin], ones, mask=m)
            return carry

        lax.fori_loop(0, _E // _L, _scatter, 0)

        pltpu.sync_copy(buf_v, out_hbm.at[pl.ds(base * _N, _ROWS * _N)])

    return _build_adj


def _cycle_body(b_ref, x_ref, out_ref):
    Bf = b_ref[...]
    bcol = (x_ref[:, 127:128] > 0.5).astype(jnp.float32)
    r_io = lax.broadcasted_iota(jnp.int32, (_N, _N), 0)
    c_io = lax.broadcasted_iota(jnp.int32, (_N, _N), 1)
    eye = (r_io == c_io).astype(jnp.float32)
    brow = jnp.sum(eye * bcol, axis=0, keepdims=True)
    P1 = jnp.minimum(Bf + eye, 1.0) * brow
    Bh = Bf.astype(jnp.bfloat16)
    T = lax.dot(Bh, P1.astype(jnp.bfloat16),
                preferred_element_type=jnp.float32)
    P2 = ((T + P1) > 0).astype(jnp.bfloat16)
    F = lax.dot(Bh, P2, preferred_element_type=jnp.float32)
    keep = (1.0 - eye) * (1.0 - Bf) * (F > 0).astype(jnp.float32)
    out_ref[...] = jnp.sum(keep, axis=1, keepdims=True).astype(jnp.int32)


_cycle_call = pl.pallas_call(
    _cycle_body,
    out_shape=jax.ShapeDtypeStruct((_N, 1), jnp.int32),
)


def kernel(x, edge_index):
    b_flat = _build_adj_kernel()(edge_index)
    b_mat = b_flat.reshape(_N, _N)
    out = _cycle_call(b_mat, x)
    return out.reshape(_N)


# speedup vs baseline: 23.4333x; 1.4026x over previous
import functools

import jax
import jax.numpy as jnp
from jax import lax
from jax.experimental import pallas as pl
from jax.experimental.pallas import tpu as pltpu
from jax.experimental.pallas import tpu_sc as plsc

_N = 1024
_E = 16384

_NC = 2
_NS = 16
_NW = _NC * _NS
_ROWS = _N // _NW
_L = 16

@functools.lru_cache(maxsize=None)
def _build_adj_kernel():
    mesh = plsc.VectorSubcoreMesh(core_axis_name="c", subcore_axis_name="s")

    @functools.partial(
        pl.kernel,
        mesh=mesh,
        out_type=jax.ShapeDtypeStruct((_N * _N,), jnp.float32),
        scratch_types=[
            pltpu.VMEM((_E,), jnp.int32),
            pltpu.VMEM((_E,), jnp.int32),
            pltpu.VMEM((_ROWS * _N,), jnp.float32),
            pltpu.SemaphoreType.DMA,
            pltpu.SemaphoreType.DMA,
        ],
        compiler_params=pltpu.CompilerParams(needs_layout_passes=False),
    )
    def _build_adj(edge_hbm, out_hbm, src_v, dst_v, buf_v, sem_s, sem_d):
        wid = lax.axis_index("s") * _NC + lax.axis_index("c")
        base = wid * _ROWS

        cp_s = pltpu.make_async_copy(edge_hbm.at[0], src_v, sem_s)
        cp_d = pltpu.make_async_copy(edge_hbm.at[1], dst_v, sem_d)
        cp_s.start()
        cp_d.start()

        zeros = jnp.zeros((_L,), jnp.float32)

        @plsc.parallel_loop(0, (_ROWS * _N) // _L, unroll=8)
        def _zero(i):
            buf_v[pl.ds(i * _L, _L)] = zeros

        cp_s.wait()
        cp_d.wait()

        ones = jnp.ones((_L,), jnp.float32)

        @plsc.parallel_loop(0, _E // _L, unroll=8)
        def _scatter(i):
            s16 = src_v[pl.ds(i * _L, _L)]
            d16 = dst_v[pl.ds(i * _L, _L)]
            rel = d16 - base
            m = rel.astype(jnp.uint32) < jnp.uint32(_ROWS)
            lin = jnp.where(m, (rel << 10) + s16, 0)
            plsc.store_scatter(buf_v, [lin], ones, mask=m)

        pltpu.sync_copy(buf_v, out_hbm.at[pl.ds(base * _N, _ROWS * _N)])

    return _build_adj


def _cycle_body(b_ref, x_ref, out_ref):
    Bf = b_ref[...]
    bcol = (x_ref[:, 127:128] > 0.5).astype(jnp.float32)
    r_io = lax.broadcasted_iota(jnp.int32, (_N, _N), 0)
    c_io = lax.broadcasted_iota(jnp.int32, (_N, _N), 1)
    eye = (r_io == c_io).astype(jnp.float32)
    brow = jnp.sum(eye * bcol, axis=0, keepdims=True)
    P1 = jnp.minimum(Bf + eye, 1.0) * brow
    Bh = Bf.astype(jnp.bfloat16)
    T = lax.dot(Bh, P1.astype(jnp.bfloat16),
                preferred_element_type=jnp.float32)
    P2 = ((T + P1) > 0).astype(jnp.bfloat16)
    F = lax.dot(Bh, P2, preferred_element_type=jnp.float32)
    keep = (1.0 - eye) * (1.0 - Bf) * (F > 0).astype(jnp.float32)
    out_ref[...] = jnp.sum(keep, axis=1, keepdims=True).astype(jnp.int32)


_cycle_call = pl.pallas_call(
    _cycle_body,
    out_shape=jax.ShapeDtypeStruct((_N, 1), jnp.int32),
)


def kernel(x, edge_index):
    b_flat = _build_adj_kernel()(edge_index)
    b_mat = b_flat.reshape(_N, _N)
    out = _cycle_call(b_mat, x)
    return out.reshape(_N)


# speedup vs baseline: 25.3293x; 1.0809x over previous
import functools

import jax
import jax.numpy as jnp
from jax import lax
from jax.experimental import pallas as pl
from jax.experimental.pallas import tpu as pltpu
from jax.experimental.pallas import tpu_sc as plsc

_N = 1024
_E = 16384

_NC = 2
_NS = 16
_NW = _NC * _NS
_ROWS = _N // _NW
_L = 16

@functools.lru_cache(maxsize=None)
def _build_adj_kernel():
    mesh = plsc.VectorSubcoreMesh(core_axis_name="c", subcore_axis_name="s")

    @functools.partial(
        pl.kernel,
        mesh=mesh,
        out_type=jax.ShapeDtypeStruct((_N, _N), jnp.float32),
        scratch_types=[
            pltpu.VMEM((_E,), jnp.int32),
            pltpu.VMEM((_E,), jnp.int32),
            pltpu.VMEM((_ROWS, _N), jnp.float32),
            pltpu.SemaphoreType.DMA,
            pltpu.SemaphoreType.DMA,
        ],
        compiler_params=pltpu.CompilerParams(needs_layout_passes=False),
    )
    def _build_adj(edge_hbm, out_hbm, src_v, dst_v, buf_v, sem_s, sem_d):
        wid = lax.axis_index("s") * _NC + lax.axis_index("c")
        base = wid * _ROWS

        cp_s = pltpu.make_async_copy(edge_hbm.at[0], src_v, sem_s)
        cp_d = pltpu.make_async_copy(edge_hbm.at[1], dst_v, sem_d)
        cp_s.start()
        cp_d.start()

        zeros = jnp.zeros((_L,), jnp.float32)

        for r in range(_ROWS):
            @plsc.parallel_loop(0, _N // _L, unroll=8)
            def _zero(i, r=r):
                buf_v[r, pl.ds(i * _L, _L)] = zeros

        cp_s.wait()
        cp_d.wait()

        ones = jnp.ones((_L,), jnp.float32)

        @plsc.parallel_loop(0, _E // _L, unroll=8)
        def _scatter(i):
            s16 = src_v[pl.ds(i * _L, _L)]
            d16 = dst_v[pl.ds(i * _L, _L)]
            rel = d16 - base
            m = rel.astype(jnp.uint32) < jnp.uint32(_ROWS)
            plsc.store_scatter(buf_v, [rel, s16], ones, mask=m)

        pltpu.sync_copy(buf_v, out_hbm.at[pl.ds(base, _ROWS)])

    return _build_adj


def _cycle_body(b_ref, x_ref, out_ref):
    Bf = b_ref[...]
    bcol = (x_ref[:, 127:128] > 0.5).astype(jnp.float32)
    r_io = lax.broadcasted_iota(jnp.int32, (_N, _N), 0)
    c_io = lax.broadcasted_iota(jnp.int32, (_N, _N), 1)
    eye = (r_io == c_io).astype(jnp.float32)
    brow = jnp.sum(eye * bcol, axis=0, keepdims=True)
    P1 = jnp.minimum(Bf + eye, 1.0) * brow
    Bh = Bf.astype(jnp.bfloat16)
    T = lax.dot(Bh, P1.astype(jnp.bfloat16),
                preferred_element_type=jnp.float32)
    P2 = ((T + P1) > 0).astype(jnp.bfloat16)
    F = lax.dot(Bh, P2, preferred_element_type=jnp.float32)
    keep = (1.0 - eye) * (1.0 - Bf) * (F > 0).astype(jnp.float32)
    out_ref[...] = jnp.sum(keep, axis=1, keepdims=True).astype(jnp.int32)


_cycle_call = pl.pallas_call(
    _cycle_body,
    out_shape=jax.ShapeDtypeStruct((_N, 1), jnp.int32),
)


def kernel(x, edge_index):
    b_mat = _build_adj_kernel()(edge_index)
    out = _cycle_call(b_mat, x)
    return out.reshape(_N)


# speedup vs baseline: 26.0724x; 1.0293x over previous
import functools

import jax
import jax.numpy as jnp
from jax import lax
from jax.experimental import pallas as pl
from jax.experimental.pallas import tpu as pltpu
from jax.experimental.pallas import tpu_sc as plsc

_N = 1024
_E = 16384

_NC = 2
_NS = 16
_NW = _NC * _NS
_ROWS = _N // _NW
_L = 16

@functools.lru_cache(maxsize=None)
def _build_adj_kernel():
    mesh = plsc.VectorSubcoreMesh(core_axis_name="c", subcore_axis_name="s")

    @functools.partial(
        pl.kernel,
        mesh=mesh,
        out_type=jax.ShapeDtypeStruct((_N, _N), jnp.float32),
        scratch_types=[
            pltpu.VMEM((_E,), jnp.int32),
            pltpu.VMEM((_E,), jnp.int32),
            pltpu.VMEM((_ROWS, _N), jnp.float32),
            pltpu.SemaphoreType.DMA,
            pltpu.SemaphoreType.DMA,
        ],
        compiler_params=pltpu.CompilerParams(needs_layout_passes=False),
    )
    def _build_adj(edge_hbm, out_hbm, src_v, dst_v, buf_v, sem_s, sem_d):
        wid = lax.axis_index("s") * _NC + lax.axis_index("c")
        base = wid * _ROWS

        cp_s = pltpu.make_async_copy(edge_hbm.at[0], src_v, sem_s)
        cp_d = pltpu.make_async_copy(edge_hbm.at[1], dst_v, sem_d)
        cp_s.start()
        cp_d.start()

        zeros = jnp.zeros((_L,), jnp.float32)

        for r in range(_ROWS):
            @plsc.parallel_loop(0, _N // _L, unroll=8)
            def _zero(i, r=r):
                buf_v[r, pl.ds(i * _L, _L)] = zeros

        cp_s.wait()
        cp_d.wait()

        ones = jnp.ones((_L,), jnp.float32)

        @plsc.parallel_loop(0, _E // _L, unroll=8)
        def _scatter(i):
            s16 = src_v[pl.ds(i * _L, _L)]
            d16 = dst_v[pl.ds(i * _L, _L)]
            rel = d16 - base
            m = rel.astype(jnp.uint32) < jnp.uint32(_ROWS)
            plsc.store_scatter(buf_v, [rel, s16], ones, mask=m)

        pltpu.sync_copy(buf_v, out_hbm.at[pl.ds(base, _ROWS)])

    return _build_adj


def _cycle_body(b_ref, x_ref, out_ref):
    Bf = b_ref[...]
    bcol = (x_ref[:, 127:128] > 0.5).astype(jnp.float32)
    r_io = lax.broadcasted_iota(jnp.int32, (_N, _N), 0)
    c_io = lax.broadcasted_iota(jnp.int32, (_N, _N), 1)
    eye = (r_io == c_io).astype(jnp.float32)
    brow = jnp.sum(eye * bcol, axis=0, keepdims=True)
    P1 = jnp.minimum(Bf + eye, 1.0) * brow
    Bh = Bf.astype(jnp.bfloat16)
    T = lax.dot(Bh, P1.astype(jnp.bfloat16),
                preferred_element_type=jnp.float32)
    P2 = ((T + P1) > 0).astype(jnp.bfloat16)
    F = lax.dot(Bh, P2, preferred_element_type=jnp.float32)
    keep = (1.0 - eye) * (1.0 - Bf) * (F > 0).astype(jnp.float32)
    out_ref[...] = jnp.sum(keep, axis=1).astype(jnp.int32)


_cycle_call = pl.pallas_call(
    _cycle_body,
    out_shape=jax.ShapeDtypeStruct((_N,), jnp.int32),
)


def kernel(x, edge_index):
    b_mat = _build_adj_kernel()(edge_index)
    return _cycle_call(b_mat, x)


# speedup vs baseline: 26.7799x; 1.0271x over previous
import functools

import jax
import jax.numpy as jnp
from jax import lax
from jax.experimental import pallas as pl
from jax.experimental.pallas import tpu as pltpu
from jax.experimental.pallas import tpu_sc as plsc

_N = 1024
_E = 16384

_NC = 2
_NS = 16
_NW = _NC * _NS
_ROWS = _N // _NW
_L = 16

@functools.lru_cache(maxsize=None)
def _build_adj_kernel():
    mesh = plsc.VectorSubcoreMesh(core_axis_name="c", subcore_axis_name="s")

    @functools.partial(
        pl.kernel,
        mesh=mesh,
        out_type=jax.ShapeDtypeStruct((_N, _N), jnp.float32),
        scratch_types=[
            pltpu.VMEM((_E,), jnp.int32),
            pltpu.VMEM((_E,), jnp.int32),
            pltpu.VMEM((_ROWS, _N), jnp.float32),
            pltpu.SemaphoreType.DMA,
            pltpu.SemaphoreType.DMA,
        ],
        compiler_params=pltpu.CompilerParams(needs_layout_passes=False),
    )
    def _build_adj(edge_hbm, out_hbm, src_v, dst_v, buf_v, sem_s, sem_d):
        wid = lax.axis_index("s") * _NC + lax.axis_index("c")
        base = wid * _ROWS

        cp_s = pltpu.make_async_copy(edge_hbm.at[0], src_v, sem_s)
        cp_d = pltpu.make_async_copy(edge_hbm.at[1], dst_v, sem_d)
        cp_s.start()
        cp_d.start()

        zeros = jnp.zeros((_L,), jnp.float32)

        @plsc.parallel_loop(0, (_ROWS * _N) // _L, unroll=8)
        def _zero(i):
            buf_v[i >> 6, pl.ds((i & 63) * _L, _L)] = zeros

        cp_s.wait()
        cp_d.wait()

        ones = jnp.ones((_L,), jnp.float32)

        @plsc.parallel_loop(0, _E // _L, unroll=8)
        def _scatter(i):
            s16 = src_v[pl.ds(i * _L, _L)]
            d16 = dst_v[pl.ds(i * _L, _L)]
            rel = d16 - base
            m = rel.astype(jnp.uint32) < jnp.uint32(_ROWS)
            plsc.store_scatter(buf_v, [rel, s16], ones, mask=m)

        pltpu.sync_copy(buf_v, out_hbm.at[pl.ds(base, _ROWS)])

    return _build_adj


def _cycle_body(b_ref, x_ref, out_ref):
    Bf = b_ref[...]
    bcol = (x_ref[:, 127:128] > 0.5).astype(jnp.float32)
    r_io = lax.broadcasted_iota(jnp.int32, (_N, _N), 0)
    c_io = lax.broadcasted_iota(jnp.int32, (_N, _N), 1)
    eye = (r_io == c_io).astype(jnp.float32)
    brow = jnp.sum(eye * bcol, axis=0, keepdims=True)
    P1 = jnp.minimum(Bf + eye, 1.0) * brow
    Bh = Bf.astype(jnp.bfloat16)
    T = lax.dot(Bh, P1.astype(jnp.bfloat16),
                preferred_element_type=jnp.float32)
    P2 = ((T + P1) > 0).astype(jnp.bfloat16)
    F = lax.dot(Bh, P2, preferred_element_type=jnp.float32)
    keep = (1.0 - eye) * (1.0 - Bf) * (F > 0).astype(jnp.float32)
    out_ref[...] = jnp.sum(keep, axis=1).astype(jnp.int32)


_cycle_call = pl.pallas_call(
    _cycle_body,
    out_shape=jax.ShapeDtypeStruct((_N,), jnp.int32),
)


def kernel(x, edge_index):
    b_mat = _build_adj_kernel()(edge_index)
    return _cycle_call(b_mat, x)


# speedup vs baseline: 27.2613x; 1.0180x over previous
import functools

import jax
import jax.numpy as jnp
from jax import lax
from jax.experimental import pallas as pl
from jax.experimental.pallas import tpu as pltpu
from jax.experimental.pallas import tpu_sc as plsc

_N = 1024
_E = 16384

_NC = 2
_NS = 16
_NW = _NC * _NS
_ROWS = _N // _NW
_L = 16

@functools.lru_cache(maxsize=None)
def _build_adj_kernel():
    mesh = plsc.VectorSubcoreMesh(core_axis_name="c", subcore_axis_name="s")

    @functools.partial(
        pl.kernel,
        mesh=mesh,
        out_type=jax.ShapeDtypeStruct((_N, _N), jnp.float32),
        scratch_types=[
            pltpu.VMEM((_E,), jnp.int32),
            pltpu.VMEM((_E,), jnp.int32),
            pltpu.VMEM((_ROWS, _N), jnp.float32),
            pltpu.SemaphoreType.DMA,
            pltpu.SemaphoreType.DMA,
        ],
        compiler_params=pltpu.CompilerParams(needs_layout_passes=False),
    )
    def _build_adj(edge_hbm, out_hbm, src_v, dst_v, buf_v, sem_s, sem_d):
        wid = lax.axis_index("s") * _NC + lax.axis_index("c")
        base = wid * _ROWS

        cp_s = pltpu.make_async_copy(edge_hbm.at[0], src_v, sem_s)
        cp_d = pltpu.make_async_copy(edge_hbm.at[1], dst_v, sem_d)
        cp_s.start()
        cp_d.start()

        zeros = jnp.zeros((_L,), jnp.float32)

        @plsc.parallel_loop(0, (_ROWS * _N) // _L, unroll=8)
        def _zero(i):
            buf_v[i >> 6, pl.ds((i & 63) * _L, _L)] = zeros

        cp_s.wait()
        cp_d.wait()

        ones = jnp.ones((_L,), jnp.float32)

        @plsc.parallel_loop(0, _E // _L, unroll=8)
        def _scatter(i):
            s16 = src_v[pl.ds(i * _L, _L)]
            d16 = dst_v[pl.ds(i * _L, _L)]
            rel = d16 - base
            m = rel.astype(jnp.uint32) < jnp.uint32(_ROWS)
            plsc.store_scatter(buf_v, [rel, s16], ones, mask=m)

        pltpu.sync_copy(buf_v, out_hbm.at[pl.ds(base, _ROWS)])

    return _build_adj


def _cycle_body(b_ref, x_ref, out_ref):
    Bf = b_ref[...]
    bcol = (x_ref[:, 127:128] > 0.5).astype(jnp.float32)
    r_io = lax.broadcasted_iota(jnp.int32, (_N, _N), 0)
    c_io = lax.broadcasted_iota(jnp.int32, (_N, _N), 1)
    eye = (r_io == c_io).astype(jnp.float32)
    brow = jnp.transpose(bcol, (1, 0))
    P1 = jnp.minimum(Bf + eye, 1.0) * brow
    Bh = Bf.astype(jnp.bfloat16)
    T = lax.dot(Bh, P1.astype(jnp.bfloat16),
                preferred_element_type=jnp.float32)
    P2 = ((T + P1) > 0).astype(jnp.bfloat16)
    F = lax.dot(Bh, P2, preferred_element_type=jnp.float32)
    keep = (F > 0) & (Bf == 0) & (r_io != c_io)
    cnt = lax.dot(keep.astype(jnp.bfloat16), jnp.ones((_N, 1), jnp.bfloat16),
                  preferred_element_type=jnp.float32)
    out_ref[...] = jnp.transpose(cnt, (1, 0)).reshape(_N).astype(jnp.int32)


_cycle_call = pl.pallas_call(
    _cycle_body,
    out_shape=jax.ShapeDtypeStruct((_N,), jnp.int32),
)


def kernel(x, edge_index):
    b_mat = _build_adj_kernel()(edge_index)
    return _cycle_call(b_mat, x)
